# Initial kernel scaffold; baseline (speedup 1.0000x reference)
import jax
import jax.numpy as jnp
from jax.experimental import pallas as pl


def kernel(feature, scales_map, scales):
    raise NotImplementedError("write your pallas kernel here")


# speedup vs baseline: 200.4626x; 200.4626x over previous
import dataclasses

import jax
import jax.numpy as jnp
from jax.experimental import pallas as pl
from jax.experimental.pallas import tpu as pltpu
from jax.experimental.pallas import tpu_sc as plsc

_B, _F, _D = 4096, 100, 64
_N = _B * _F * _D
_NUM_SCALES = 100000
_CHUNK = 2048
_GRID = _N // _CHUNK
_LANES = 16


def _sc_kernel_body(map_hbm, feat_hbm, scales_hbm, out_hbm, table_vmem):
    pltpu.sync_copy(scales_hbm, table_vmem)

    def chunk_body(map_vmem, feat_vmem, out_vmem):
        @pl.loop(0, _CHUNK, step=_LANES)
        def _(j):
            idx = map_vmem[pl.ds(j, _LANES)]
            vals = plsc.load_gather(table_vmem, [idx])
            out_vmem[pl.ds(j, _LANES)] = feat_vmem[pl.ds(j, _LANES)] * vals

    pltpu.emit_pipeline(
        chunk_body,
        grid=(_GRID,),
        in_specs=[
            pl.BlockSpec((_CHUNK,), lambda i: (i,)),
            pl.BlockSpec((_CHUNK,), lambda i: (i,)),
        ],
        out_specs=[pl.BlockSpec((_CHUNK,), lambda i: (i,))],
        core_axis_name=("core", "subcore"),
        dimension_semantics=(pltpu.PARALLEL,),
    )(map_hbm, feat_hbm, out_hbm)


@jax.jit
def kernel(feature, scales_map, scales):
    mesh = plsc.VectorSubcoreMesh(
        core_axis_name="core", subcore_axis_name="subcore"
    )
    cp = pltpu.CompilerParams()
    if "needs_layout_passes" in pltpu.CompilerParams.__dataclass_fields__:
        cp = dataclasses.replace(cp, needs_layout_passes=False)
    run = pl.kernel(
        _sc_kernel_body,
        out_type=jax.ShapeDtypeStruct((_N,), jnp.float32),
        mesh=mesh,
        scratch_types=[pltpu.VMEM((_NUM_SCALES,), jnp.float32)],
        compiler_params=cp,
    )
    out_flat = run(scales_map.reshape(_N), feature.reshape(_N), scales)
    return out_flat.reshape(_B, _F, _D)


# speedup vs baseline: 204.7861x; 1.0216x over previous
import dataclasses

import jax
import jax.numpy as jnp
from jax.experimental import pallas as pl
from jax.experimental.pallas import tpu as pltpu
from jax.experimental.pallas import tpu_sc as plsc

_B, _F, _D = 4096, 100, 64
_N = _B * _F * _D
_NUM_SCALES = 100000
_CHUNK = 4096
_GRID = _N // _CHUNK
_LANES = 16


def _sc_kernel_body(map_hbm, feat_hbm, scales_hbm, out_hbm, table_vmem):
    pltpu.sync_copy(scales_hbm, table_vmem)

    def chunk_body(map_vmem, feat_vmem, out_vmem):
        @pl.loop(0, _CHUNK, step=_LANES * 8)
        def _(j):
            for u in range(8):
                o = j + u * _LANES
                idx = map_vmem[pl.ds(o, _LANES)]
                vals = plsc.load_gather(table_vmem, [idx])
                out_vmem[pl.ds(o, _LANES)] = feat_vmem[pl.ds(o, _LANES)] * vals

    pltpu.emit_pipeline(
        chunk_body,
        grid=(_GRID,),
        in_specs=[
            pl.BlockSpec((_CHUNK,), lambda i: (i,)),
            pl.BlockSpec((_CHUNK,), lambda i: (i,)),
        ],
        out_specs=[pl.BlockSpec((_CHUNK,), lambda i: (i,))],
        core_axis_name=("core", "subcore"),
        dimension_semantics=(pltpu.PARALLEL,),
    )(map_hbm, feat_hbm, out_hbm)


@jax.jit
def kernel(feature, scales_map, scales):
    mesh = plsc.VectorSubcoreMesh(
        core_axis_name="core", subcore_axis_name="subcore"
    )
    cp = pltpu.CompilerParams()
    if "needs_layout_passes" in pltpu.CompilerParams.__dataclass_fields__:
        cp = dataclasses.replace(cp, needs_layout_passes=False)
    run = pl.kernel(
        _sc_kernel_body,
        out_type=jax.ShapeDtypeStruct((_N,), jnp.float32),
        mesh=mesh,
        scratch_types=[pltpu.VMEM((_NUM_SCALES,), jnp.float32)],
        compiler_params=cp,
    )
    out_flat = run(scales_map.reshape(_N), feature.reshape(_N), scales)
    return out_flat.reshape(_B, _F, _D)


# speedup vs baseline: 461.1890x; 2.2521x over previous
import dataclasses

import jax
import jax.numpy as jnp
from jax.experimental import pallas as pl
from jax.experimental.pallas import tpu as pltpu
from jax.experimental.pallas import tpu_sc as plsc

_B, _F, _D = 4096, 100, 64
_NUM_SCALES = 100000
_CHUNK = 4096
_LANES = 16


def _sc_kernel_body(map_hbm, feat_hbm, scales_hbm, out_hbm, table_vmem):
    pltpu.sync_copy(scales_hbm, table_vmem)

    def chunk_body(map_vmem, feat_vmem, out_vmem):
        @pl.loop(0, _CHUNK, step=_LANES * 8)
        def _(j):
            for u in range(8):
                o = j + u * _LANES
                idx = map_vmem[0, 0, pl.ds(o, _LANES)]
                vals = plsc.load_gather(table_vmem, [idx])
                out_vmem[0, 0, pl.ds(o, _LANES)] = (
                    feat_vmem[0, 0, pl.ds(o, _LANES)] * vals
                )

    pltpu.emit_pipeline(
        chunk_body,
        grid=(_F, _D),
        in_specs=[
            pl.BlockSpec((1, 1, _B), lambda i, j: (i, j, 0)),
            pl.BlockSpec((1, 1, _B), lambda i, j: (i, j, 0)),
        ],
        out_specs=[pl.BlockSpec((1, 1, _B), lambda i, j: (i, j, 0))],
        core_axis_name=("core", "subcore"),
        dimension_semantics=(pltpu.PARALLEL, pltpu.PARALLEL),
    )(map_hbm, feat_hbm, out_hbm)


@jax.jit
def kernel(feature, scales_map, scales):
    mesh = plsc.VectorSubcoreMesh(
        core_axis_name="core", subcore_axis_name="subcore"
    )
    cp = pltpu.CompilerParams()
    if "needs_layout_passes" in pltpu.CompilerParams.__dataclass_fields__:
        cp = dataclasses.replace(cp, needs_layout_passes=False)
    run = pl.kernel(
        _sc_kernel_body,
        out_type=jax.ShapeDtypeStruct((_F, _D, _B), jnp.float32),
        mesh=mesh,
        scratch_types=[pltpu.VMEM((_NUM_SCALES,), jnp.float32)],
        compiler_params=cp,
    )
    map_t = jnp.transpose(scales_map, (1, 2, 0))
    feat_t = jnp.transpose(feature, (1, 2, 0))
    out_t = run(map_t, feat_t, scales)
    return jnp.transpose(out_t, (2, 0, 1))


# speedup vs baseline: 1626.8570x; 3.5275x over previous
import dataclasses

import jax
import jax.numpy as jnp
from jax.experimental import pallas as pl
from jax.experimental.pallas import tpu as pltpu
from jax.experimental.pallas import tpu_sc as plsc

_B, _F, _D = 4096, 100, 64
_NUM_SCALES = 100000
_CHUNK = 4096
_LANES = 16


def _sc_kernel_body(map_hbm, feat_hbm, scales_hbm, out_hbm, table_vmem):
    pltpu.sync_copy(scales_hbm, table_vmem)

    def chunk_body(map_vmem, feat_vmem, out_vmem):
        @plsc.parallel_loop(0, _CHUNK, step=_LANES, unroll=8)
        def _(j):
            idx = map_vmem[0, 0, pl.ds(j, _LANES)]
            vals = plsc.load_gather(table_vmem, [idx])
            out_vmem[0, 0, pl.ds(j, _LANES)] = (
                feat_vmem[0, 0, pl.ds(j, _LANES)] * vals
            )

    pltpu.emit_pipeline(
        chunk_body,
        grid=(_F, _D),
        in_specs=[
            pl.BlockSpec((1, 1, _B), lambda i, j: (i, j, 0)),
            pl.BlockSpec((1, 1, _B), lambda i, j: (i, j, 0)),
        ],
        out_specs=[pl.BlockSpec((1, 1, _B), lambda i, j: (i, j, 0))],
        core_axis_name=("core", "subcore"),
        dimension_semantics=(pltpu.PARALLEL, pltpu.PARALLEL),
    )(map_hbm, feat_hbm, out_hbm)


@jax.jit
def kernel(feature, scales_map, scales):
    mesh = plsc.VectorSubcoreMesh(
        core_axis_name="core", subcore_axis_name="subcore"
    )
    cp = pltpu.CompilerParams()
    if "needs_layout_passes" in pltpu.CompilerParams.__dataclass_fields__:
        cp = dataclasses.replace(cp, needs_layout_passes=False)
    run = pl.kernel(
        _sc_kernel_body,
        out_type=jax.ShapeDtypeStruct((_F, _D, _B), jnp.float32),
        mesh=mesh,
        scratch_types=[pltpu.VMEM((_NUM_SCALES,), jnp.float32)],
        compiler_params=cp,
    )
    map_t = jnp.transpose(scales_map, (1, 2, 0))
    feat_t = jnp.transpose(feature, (1, 2, 0))
    out_t = run(map_t, feat_t, scales)
    return jnp.transpose(out_t, (2, 0, 1))
